# Initial kernel scaffold; baseline (speedup 1.0000x reference)
import jax
import jax.numpy as jnp
from jax.experimental import pallas as pl


def kernel(coords, pairs, box, charges, coulomb_constant, cutoff):
    raise NotImplementedError("write your pallas kernel here")


# speedup vs baseline: 54.7282x; 54.7282x over previous
import functools
import math

import numpy as np
import jax
import jax.numpy as jnp
from jax import lax
from jax.experimental import pallas as pl
from jax.experimental.pallas import tpu as pltpu
from jax.experimental.pallas import tpu_sc as plsc

NC = 2
NS = 16
NW = NC * NS
LANES = 16
BLK = 4096


def _sc_coulomb(n_pairs, nb):
    mesh = plsc.VectorSubcoreMesh(core_axis_name="c", subcore_axis_name="s")

    @functools.partial(
        pl.kernel,
        out_type=jax.ShapeDtypeStruct((NW, LANES), jnp.float32),
        mesh=mesh,
        scratch_types=[
            pltpu.VMEM((BLK,), jnp.int32),
            pltpu.VMEM((BLK,), jnp.int32),
            pltpu.VMEM((BLK,), jnp.float32),
            pltpu.VMEM((BLK,), jnp.float32),
            pltpu.VMEM((BLK,), jnp.float32),
            pltpu.VMEM((BLK,), jnp.float32),
            pltpu.VMEM((BLK,), jnp.float32),
            pltpu.VMEM((BLK,), jnp.float32),
            pltpu.VMEM((BLK,), jnp.float32),
            pltpu.VMEM((BLK,), jnp.float32),
            pltpu.VMEM((32,), jnp.float32),
            pltpu.VMEM((LANES,), jnp.float32),
            pltpu.SemaphoreType.DMA,
        ],
    )
    def kern(tx_hbm, ty_hbm, tz_hbm, tq_hbm, pi_hbm, pj_hbm, par_hbm,
             out_hbm, idxi, idxj, bxi, byi, bzi, bqi, bxj, byj, bzj, bqj,
             par_v, accv, sem):
        cid = lax.axis_index("c")
        sid = lax.axis_index("s")
        wid = sid * NC + cid

        pltpu.sync_copy(par_hbm, par_v)
        va = par_v[pl.ds(0, LANES)]
        vb = par_v[pl.ds(LANES, LANES)]
        b00, b01, b02 = va[0], va[1], va[2]
        b10, b11, b12 = va[3], va[4], va[5]
        b20, b21, b22 = va[6], va[7], va[8]
        i00, i01, i02 = va[9], va[10], va[11]
        i10, i11, i12 = va[12], va[13], va[14]
        i20, i21, i22 = va[15], vb[0], vb[1]
        cut2, inv_cut = vb[2], vb[3]

        iota = lax.broadcasted_iota(jnp.int32, (LANES,), 0)
        wbase = wid * (nb * BLK)
        bmask = jnp.int32(np.int32(np.uint32(0xFFFF0000)))

        def bq(v):
            b = lax.bitcast_convert_type(v, jnp.int32)
            b = b + 0x7FFF + (lax.shift_right_logical(b, 16) & 1)
            b = b & bmask
            return lax.bitcast_convert_type(b, jnp.float32)

        def block_body(g, acc):
            base = wbase + g * BLK
            pltpu.sync_copy(pi_hbm.at[pl.ds(base, BLK)], idxi)
            pltpu.sync_copy(pj_hbm.at[pl.ds(base, BLK)], idxj)
            cps = [
                pltpu.async_copy(tx_hbm.at[idxi], bxi, sem),
                pltpu.async_copy(ty_hbm.at[idxi], byi, sem),
                pltpu.async_copy(tz_hbm.at[idxi], bzi, sem),
                pltpu.async_copy(tq_hbm.at[idxi], bqi, sem),
                pltpu.async_copy(tx_hbm.at[idxj], bxj, sem),
                pltpu.async_copy(ty_hbm.at[idxj], byj, sem),
                pltpu.async_copy(tz_hbm.at[idxj], bzj, sem),
                pltpu.async_copy(tq_hbm.at[idxj], bqj, sem),
            ]
            for cp in cps:
                cp.wait()

            def group_body(g2, acc2):
                off = pl.ds(g2 * LANES, LANES)
                xi, yi, zi, qi = bxi[off], byi[off], bzi[off], bqi[off]
                xj, yj, zj, qj = bxj[off], byj[off], bzj[off], bqj[off]

                dx = bq(xi - xj)
                dy = bq(yi - yj)
                dz = bq(zi - zj)
                s0 = dx * i00 + dy * i10 + dz * i20
                s1 = dx * i01 + dy * i11 + dz * i21
                s2 = dx * i02 + dy * i12 + dz * i22

                def wrap(s):
                    u = s + 0.5
                    t = u.astype(jnp.int32).astype(jnp.float32)
                    fl = jnp.where(t > u, t - 1.0, t)
                    return s - fl

                s0 = bq(wrap(s0))
                s1 = bq(wrap(s1))
                s2 = bq(wrap(s2))

                e0 = s0 * b00 + s1 * b10 + s2 * b20
                e1 = s0 * b01 + s1 * b11 + s2 * b21
                e2 = s0 * b02 + s1 * b12 + s2 * b22
                r2 = e0 * e0 + e1 * e1 + e2 * e2

                ibits = lax.bitcast_convert_type(r2, jnp.int32)
                seed = jnp.int32(0x5F3759DF) - lax.shift_right_logical(
                    ibits, jnp.int32(1))
                y = lax.bitcast_convert_type(seed, jnp.float32)
                xh = 0.5 * r2
                y = y * (1.5 - xh * y * y)
                y = y * (1.5 - xh * y * y)
                y = y * (1.5 - xh * y * y)

                ene = (qi * qj) * (y - inv_cut)
                gidx = base + g2 * LANES + iota
                keep = jnp.logical_and(r2 <= cut2, gidx < n_pairs)
                return acc2 + jnp.where(keep, ene, 0.0)

            return lax.fori_loop(0, BLK // LANES, group_body, acc)

        acc = lax.fori_loop(0, nb, block_body,
                            jnp.zeros((LANES,), jnp.float32))
        accv[...] = acc
        pltpu.sync_copy(accv, out_hbm.at[wid])

    return kern


def kernel(coords, pairs, box, charges, coulomb_constant, cutoff):
    m = pairs.shape[0]
    nb = math.ceil(m / (NW * BLK))
    m_pad = NW * nb * BLK

    tx = coords[:, 0].astype(jnp.float32)
    ty = coords[:, 1].astype(jnp.float32)
    tz = coords[:, 2].astype(jnp.float32)
    tq = charges.astype(jnp.float32)
    pi = pairs[:, 0].astype(jnp.int32)
    pj = pairs[:, 1].astype(jnp.int32)
    pad = m_pad - m
    pi = jnp.concatenate([pi, jnp.zeros((pad,), jnp.int32)])
    pj = jnp.concatenate([pj, jnp.zeros((pad,), jnp.int32)])

    binv = jnp.linalg.inv(box)
    box_q = box.astype(jnp.bfloat16).astype(jnp.float32)
    binv_q = binv.astype(jnp.bfloat16).astype(jnp.float32)
    cutf = jnp.asarray(cutoff, jnp.float32)
    params = jnp.concatenate(
        [box_q.reshape(-1), binv_q.reshape(-1),
         jnp.stack([cutf * cutf, 1.0 / cutf]),
         jnp.zeros((12,), jnp.float32)]).astype(jnp.float32)

    part = _sc_coulomb(m, nb)(tx, ty, tz, tq, pi, pj, params)
    return jnp.sum(part) * coulomb_constant
